# baseline (speedup 1.0000x reference)
import functools

import jax
import jax.numpy as jnp
from jax.experimental import pallas as pl
from jax.experimental.pallas import tpu as pltpu

_N_TOK = 16384
_K = 8192
_D = 256
_TILE_T = 256
_GRID = _N_TOK // _TILE_T
_COMMITMENT_COST = 0.25


def _vq_kernel(z_ref, cb_ref, z2_ref, cb2_ref, q_ref, idx_ref, counts_ref,
               loss_ref):
    zb = z_ref[0]
    zt = zb.T
    cb = cb_ref[...]

    z2 = z2_ref[0].reshape(_TILE_T, 1)
    s = jax.lax.dot_general(
        zt.astype(jnp.bfloat16), cb.astype(jnp.bfloat16),
        (((1,), (1,)), ((), ())),
        preferred_element_type=jnp.float32)
    dist = (z2 + cb2_ref[...]) - 2.0 * s

    _CH = 128
    lane = jax.lax.broadcasted_iota(jnp.int32, (_TILE_T, _CH), 1)
    inf = jnp.float32(jnp.inf)

    def _chunk(c):
        return dist[:, c * _CH:(c + 1) * _CH]

    def _range_min(c0, c1):
        m = _chunk(c0)
        for c in range(c0 + 1, c1):
            m = jnp.minimum(m, _chunk(c))
        return m

    b21 = _chunk(21)
    b42 = _chunk(42)
    m0 = jnp.minimum(_range_min(0, 21), jnp.where(lane < 48, b21, inf))
    m1 = jnp.minimum(
        jnp.minimum(_range_min(22, 42), jnp.where(lane >= 48, b21, inf)),
        jnp.where(lane < 96, b42, inf))
    m2 = jnp.minimum(_range_min(43, 64), jnp.where(lane >= 96, b42, inf))
    mn0 = jnp.min(m0, axis=1, keepdims=True)
    mn1 = jnp.min(m1, axis=1, keepdims=True)
    mn2 = jnp.min(m2, axis=1, keepdims=True)

    q0 = mn0.astype(jnp.bfloat16).astype(jnp.float32)
    upd1 = mn1 < q0
    g_sel = jnp.where(upd1, 1, 0)
    sel_val = jnp.where(upd1, mn1, mn0)
    q1 = jnp.where(upd1, mn1, q0).astype(jnp.bfloat16).astype(jnp.float32)
    upd2 = mn2 < q1
    g_sel = jnp.where(upd2, 2, g_sel)
    sel_val = jnp.where(upd2, mn2, sel_val)

    lo_sel = jnp.where(g_sel == 0, 0, jnp.where(g_sel == 1, 2736, 5472))
    hi_sel = jnp.where(g_sel == 0, 2736, jnp.where(g_sel == 1, 5472, _K))
    iota_k = jax.lax.broadcasted_iota(jnp.int32, (_TILE_T, _K), 1)
    eq = (dist == sel_val) & (iota_k >= lo_sel) & (iota_k < hi_sel)
    cand = jnp.where(eq, iota_k, _K)
    im = cand[:, :_CH]
    for c in range(1, _K // _CH):
        im = jnp.minimum(im, cand[:, c * _CH:(c + 1) * _CH])
    idx = jnp.min(im, axis=1, keepdims=True)

    idx_ref[0, 0, :] = idx[:, 0]

    onehot = (iota_k == idx).astype(jnp.float32)
    counts_ref[0] = jnp.sum(onehot, axis=0, keepdims=True)

    qt = jax.lax.dot_general(
        onehot, cb, (((1,), (0,)), ((), ())),
        preferred_element_type=jnp.float32)
    q_ref[0] = qt.T

    diff = zt - qt
    loss_ref[0] = jnp.full((1, 128), jnp.sum(diff * diff),
                           dtype=jnp.float32)


def _finalize_kernel(counts_ref, loss_ref, loss_out_ref, perp_out_ref):
    loss_sum = jnp.sum(loss_ref[...][:, 0, 0])
    loss = _COMMITMENT_COST * (loss_sum / float(_N_TOK * _D))
    loss_out_ref[...] = jnp.full((8, 128), loss, dtype=jnp.float32)
    counts = jnp.sum(counts_ref[...][:, 0, :], axis=0,
                     keepdims=True)
    p = counts * (1.0 / float(_N_TOK))
    ent = -jnp.sum(p * jnp.log(p + 1e-10))
    perp_out_ref[...] = jnp.full((8, 128), jnp.exp(ent), dtype=jnp.float32)


@functools.partial(jax.jit, static_argnames=("interpret",))
def _vq(z, codebook, interpret=False):
    zr = z.reshape(16, 256, 1024)
    zp = jnp.transpose(z, (0, 2, 3, 1))
    z_flat = zp.reshape(-1, _D)
    z2 = jnp.sum(z_flat ** 2, axis=1, keepdims=True).reshape(_GRID, 1, _TILE_T)
    cb2 = jnp.sum(codebook ** 2, axis=1).reshape(1, _K)
    q, idx, counts, loss_part = pl.pallas_call(
        _vq_kernel,
        grid=(_GRID,),
        in_specs=[
            pl.BlockSpec((1, _D, _TILE_T), lambda i: (i // 4, 0, i % 4)),
            pl.BlockSpec((_K, _D), lambda i: (0, 0)),
            pl.BlockSpec((1, 1, _TILE_T), lambda i: (i, 0, 0)),
            pl.BlockSpec((1, _K), lambda i: (0, 0)),
        ],
        out_specs=[
            pl.BlockSpec((1, _D, _TILE_T), lambda i: (i // 4, 0, i % 4)),
            pl.BlockSpec((1, 1, _TILE_T), lambda i: (i, 0, 0)),
            pl.BlockSpec((1, 1, _K), lambda i: (i, 0, 0)),
            pl.BlockSpec((1, 1, 128), lambda i: (i, 0, 0)),
        ],
        out_shape=[
            jax.ShapeDtypeStruct((16, 256, 1024), jnp.float32),
            jax.ShapeDtypeStruct((_GRID, 1, _TILE_T), jnp.int32),
            jax.ShapeDtypeStruct((_GRID, 1, _K), jnp.float32),
            jax.ShapeDtypeStruct((_GRID, 1, 128), jnp.float32),
        ],
        compiler_params=pltpu.CompilerParams(
            dimension_semantics=("parallel",),
        ),
        interpret=interpret,
    )(zr, codebook, z2, cb2)
    loss, perp = pl.pallas_call(
        _finalize_kernel,
        out_shape=[
            jax.ShapeDtypeStruct((8, 128), jnp.float32),
            jax.ShapeDtypeStruct((8, 128), jnp.float32),
        ],
        interpret=interpret,
    )(counts, loss_part)
    return q, loss, perp, idx


def kernel(z, codebook):
    z = z.astype(jnp.float32)
    q, loss, perp, _ = _vq(z, codebook)
    quantized = q.reshape(16, 256, 32, 32)
    return quantized, loss[0, 0], perp[0, 0]


# speedup vs baseline: 1.0954x; 1.0954x over previous
import functools

import jax
import jax.numpy as jnp
from jax.experimental import pallas as pl
from jax.experimental.pallas import tpu as pltpu

_N_TOK = 16384
_K = 8192
_D = 256
_TILE_T = 256
_GRID = _N_TOK // _TILE_T
_COMMITMENT_COST = 0.25


def _vq_kernel(z_ref, cb_ref, z2_ref, cb2_ref, q_ref, loss_ref, perp_ref,
               idx_ref, counts_ref, loss_acc_ref):
    i = pl.program_id(0)

    @pl.when(i == 0)
    def _init():
        counts_ref[...] = jnp.zeros_like(counts_ref)
        loss_acc_ref[0, 0] = 0.0

    zb = z_ref[0]
    zt = zb.T
    cb = cb_ref[...]

    z2 = z2_ref[0].reshape(_TILE_T, 1)
    s = jax.lax.dot_general(
        zt.astype(jnp.bfloat16), cb.astype(jnp.bfloat16),
        (((1,), (1,)), ((), ())),
        preferred_element_type=jnp.float32)
    dist = (z2 + cb2_ref[...]) - 2.0 * s

    iota_k = jax.lax.broadcasted_iota(jnp.int32, (_TILE_T, _K), 1)
    acc_v = jnp.full((_TILE_T, 1), jnp.inf, dtype=jnp.float32)
    acc_i = jnp.zeros((_TILE_T, 1), dtype=jnp.int32)
    for lo, hi in ((0, 2736), (2736, 5472), (5472, _K)):
        in_g = (iota_k >= lo) & (iota_k < hi)
        gd = jnp.where(in_g, dist, jnp.inf)
        mn_g = jnp.min(gd, axis=1, keepdims=True)
        id_g = jnp.min(jnp.where(gd == mn_g, iota_k, _K), axis=1,
                       keepdims=True)
        upd = mn_g < acc_v
        acc_i = jnp.where(upd, id_g, acc_i)
        acc_v = jnp.where(upd, mn_g, acc_v)
        acc_v = acc_v.astype(jnp.bfloat16).astype(jnp.float32)
    idx = acc_i

    idx_ref[0, 0, :] = idx[:, 0]

    onehot = (iota_k == idx).astype(jnp.float32)
    counts_ref[...] += jnp.sum(onehot, axis=0, keepdims=True)

    qt = jax.lax.dot_general(
        onehot, cb, (((1,), (0,)), ((), ())),
        preferred_element_type=jnp.float32)
    q_ref[0] = qt.T

    diff = zt - qt
    loss_acc_ref[0, 0] += jnp.sum(diff * diff)

    @pl.when(i == _GRID - 1)
    def _finish():
        loss = _COMMITMENT_COST * (loss_acc_ref[0, 0] / float(_N_TOK * _D))
        loss_ref[...] = jnp.full((8, 128), loss, dtype=jnp.float32)
        p = counts_ref[...] * (1.0 / float(_N_TOK))
        ent = -jnp.sum(p * jnp.log(p + 1e-10))
        perp_ref[...] = jnp.full((8, 128), jnp.exp(ent), dtype=jnp.float32)


@functools.partial(jax.jit, static_argnames=("interpret",))
def _vq(z, codebook, interpret=False):
    zr = z.reshape(16, 256, 1024)
    zp = jnp.transpose(z, (0, 2, 3, 1))
    z_flat = zp.reshape(-1, _D)
    z2 = jnp.sum(z_flat ** 2, axis=1, keepdims=True).reshape(_GRID, 1, _TILE_T)
    cb2 = jnp.sum(codebook ** 2, axis=1).reshape(1, _K)
    q, loss, perp, idx = pl.pallas_call(
        _vq_kernel,
        grid=(_GRID,),
        in_specs=[
            pl.BlockSpec((1, _D, _TILE_T), lambda i: (i // 4, 0, i % 4)),
            pl.BlockSpec((_K, _D), lambda i: (0, 0)),
            pl.BlockSpec((1, 1, _TILE_T), lambda i: (i, 0, 0)),
            pl.BlockSpec((1, _K), lambda i: (0, 0)),
        ],
        out_specs=[
            pl.BlockSpec((1, _D, _TILE_T), lambda i: (i // 4, 0, i % 4)),
            pl.BlockSpec((8, 128), lambda i: (0, 0)),
            pl.BlockSpec((8, 128), lambda i: (0, 0)),
            pl.BlockSpec((1, 1, _TILE_T), lambda i: (i, 0, 0)),
        ],
        out_shape=[
            jax.ShapeDtypeStruct((16, 256, 1024), jnp.float32),
            jax.ShapeDtypeStruct((8, 128), jnp.float32),
            jax.ShapeDtypeStruct((8, 128), jnp.float32),
            jax.ShapeDtypeStruct((_GRID, 1, _TILE_T), jnp.int32),
        ],
        scratch_shapes=[
            pltpu.VMEM((1, _K), jnp.float32),
            pltpu.SMEM((1, 1), jnp.float32),
        ],
        compiler_params=pltpu.CompilerParams(
            dimension_semantics=("arbitrary",),
        ),
        interpret=interpret,
    )(zr, codebook, z2, cb2)
    return q, loss, perp, idx


def kernel(z, codebook):
    z = z.astype(jnp.float32)
    q, loss, perp, _ = _vq(z, codebook)
    quantized = q.reshape(16, 256, 32, 32)
    return quantized, loss[0, 0], perp[0, 0]
